# Initial kernel scaffold; baseline (speedup 1.0000x reference)
import jax
import jax.numpy as jnp
from jax.experimental import pallas as pl


def kernel(node_features, edge_index, edge_vectors, W_weight, W_bias):
    raise NotImplementedError("write your pallas kernel here")


# speedup vs baseline: 2.2257x; 2.2257x over previous
import functools
from math import factorial, sqrt

import jax
import jax.numpy as jnp
import numpy as np
from jax import lax
from jax.experimental import pallas as pl
from jax.experimental.pallas import tpu as pltpu

MUL = 32
LMAX = 3
IR_DIMS = [2 * l + 1 for l in range(LMAX + 1)]
NODE_DIM = MUL * sum(IR_DIMS)
_PATHS = [(0, 0, 0), (0, 1, 1), (0, 2, 2), (0, 3, 3), (1, 0, 1), (1, 1, 0),
          (1, 1, 2), (1, 2, 1), (1, 2, 3), (1, 3, 2), (2, 0, 2), (2, 1, 1),
          (2, 1, 3), (2, 2, 0), (2, 2, 2), (2, 3, 1), (2, 3, 3), (3, 0, 3),
          (3, 1, 2), (3, 2, 1), (3, 2, 3), (3, 3, 0), (3, 3, 2)]
_OFFS = np.cumsum([0] + [MUL * d for d in IR_DIMS])


def _cg(l1, m1, l2, m2, l3, m3):
    if m1 + m2 != m3:
        return 0.0
    pref = sqrt((2 * l3 + 1) * factorial(l3 + l1 - l2) * factorial(l3 - l1 + l2)
                * factorial(l1 + l2 - l3) / factorial(l1 + l2 + l3 + 1))
    pref *= sqrt(factorial(l3 + m3) * factorial(l3 - m3) * factorial(l1 - m1)
                 * factorial(l1 + m1) * factorial(l2 - m2) * factorial(l2 + m2))
    s = 0.0
    for k in range(0, l1 + l2 + l3 + 1):
        d = [k, l1 + l2 - l3 - k, l1 - m1 - k, l2 + m2 - k,
             l3 - l2 + m1 + k, l3 - l1 - m2 + k]
        if min(d) < 0:
            continue
        den = 1.0
        for t in d:
            den *= factorial(t)
        s += (-1) ** k / den
    return pref * s


def _w3j_real(l1, l2, l3):
    C = np.zeros((2 * l1 + 1, 2 * l2 + 1, 2 * l3 + 1), dtype=complex)
    for m1 in range(-l1, l1 + 1):
        for m2 in range(-l2, l2 + 1):
            for m3 in range(-l3, l3 + 1):
                C[l1 + m1, l2 + m2, l3 + m3] = _cg(l1, m1, l2, m2, l3, m3)

    def basis(l):
        U = np.zeros((2 * l + 1, 2 * l + 1), dtype=complex)
        s2 = 1.0 / sqrt(2.0)
        U[l, l] = 1.0
        for m in range(1, l + 1):
            U[l + m, l + m] = (-1) ** m * s2
            U[l + m, l - m] = s2
            U[l - m, l - m] = 1j * s2
            U[l - m, l + m] = -1j * ((-1) ** m) * s2
        return U

    U1, U2, U3 = basis(l1), basis(l2), basis(l3)
    Cr = np.einsum('am,bn,co,mno->abc', U1, U2, np.conj(U3), C)
    R = np.real(Cr)
    if np.linalg.norm(R) < 1e-6:
        R = np.imag(Cr)
    return (R / np.linalg.norm(R)).astype(np.float32)


_W3J = {p: _w3j_real(*p) for p in _PATHS}
_FANIN = {l3: MUL * sum(1 for p in _PATHS if p[2] == l3) for l3 in range(LMAX + 1)}
_ALPHA = {l3: float(np.sqrt((2 * l3 + 1) / _FANIN[l3])) for l3 in range(LMAX + 1)}
_PATHS_BY_L3 = {l3: [p for p in _PATHS if p[2] == l3] for l3 in range(LMAX + 1)}
_PATH_OFF = {p: 1024 * i for i, p in enumerate(_PATHS)}

_PERM_T2R = np.zeros(NODE_DIM, dtype=np.int32)
for _l in range(LMAX + 1):
    _d = 2 * _l + 1
    for _w in range(MUL):
        for _k in range(_d):
            _PERM_T2R[_OFFS[_l] + _w * _d + _k] = _OFFS[_l] + _k * MUL + _w

_EBLK = 256


def _dense_body(ev_ref, x_ref, w0_ref, w1_ref, w2_ref, w3_ref, out_ref):
    wrefs = [w0_ref, w1_ref, w2_ref, w3_ref]
    vx = ev_ref[:, 0:1]
    vy = ev_ref[:, 1:2]
    vz = ev_ref[:, 2:3]
    r2 = vx * vx + vy * vy + vz * vz
    length = jnp.maximum(jnp.sqrt(r2), 1e-8)
    inv = 1.0 / length
    x = vx * inv
    y = vy * inv
    z = vz * inv

    Y = {0: [jnp.ones_like(x)]}
    Y[1] = [sqrt(3.0) * y, sqrt(3.0) * z, sqrt(3.0) * x]
    Y[2] = [sqrt(15.0) * x * y, sqrt(15.0) * y * z,
            sqrt(5.0) / 2.0 * (3.0 * z * z - 1.0), sqrt(15.0) * x * z,
            sqrt(15.0) / 2.0 * (x * x - y * y)]
    Y[3] = [sqrt(35.0 / 8.0) * y * (3.0 * x * x - y * y),
            sqrt(105.0) * x * y * z,
            sqrt(21.0 / 8.0) * y * (5.0 * z * z - 1.0),
            sqrt(7.0) / 2.0 * (5.0 * z * z * z - 3.0 * z),
            sqrt(21.0 / 8.0) * x * (5.0 * z * z - 1.0),
            sqrt(105.0) / 2.0 * z * (x * x - y * y),
            sqrt(35.0 / 8.0) * x * (x * x - 3.0 * y * y)]

    def xsl(l, i):
        c = int(_OFFS[l]) + i * MUL
        return x_ref[:, c:c + MUL]

    for l3 in range(LMAX + 1):
        d3 = 2 * l3 + 1
        paths = _PATHS_BY_L3[l3]
        t_ks = []
        for k in range(d3):
            cols = []
            for (l1, l2, _l3) in paths:
                d1 = 2 * l1 + 1
                C = _W3J[(l1, l2, _l3)]
                acc = None
                for i in range(d1):
                    ycg = None
                    for j in range(2 * l2 + 1):
                        cv = float(C[i, j, k])
                        if cv == 0.0:
                            continue
                        term = cv * Y[l2][j]
                        ycg = term if ycg is None else ycg + term
                    if ycg is None:
                        continue
                    term = xsl(l1, i) * ycg
                    acc = term if acc is None else acc + term
                if acc is None:
                    acc = jnp.zeros((_EBLK, MUL), jnp.float32)
                cols.append(acc)
            t_ks.append(jnp.concatenate(cols, axis=1))
        tall = jnp.concatenate(t_ks, axis=0)
        lenf = jnp.concatenate([length] * d3, axis=0)
        tcat = jnp.concatenate([tall * lenf, tall], axis=1)
        o = lax.dot_general(tcat, wrefs[l3][...],
                            (((1,), (0,)), ((), ())),
                            preferred_element_type=jnp.float32)
        base = int(_OFFS[l3])
        for k in range(d3):
            out_ref[:, base + k * MUL:base + (k + 1) * MUL] = \
                o[k * _EBLK:(k + 1) * _EBLK, :]


def _dense_call(ev, xs, wstacks, interpret=False):
    ep = ev.shape[0]
    grid = ep // _EBLK
    wspecs = [pl.BlockSpec(w.shape, lambda i: (0, 0)) for w in wstacks]
    return pl.pallas_call(
        _dense_body,
        grid=(grid,),
        in_specs=[pl.BlockSpec((_EBLK, 3), lambda i: (i, 0)),
                  pl.BlockSpec((_EBLK, NODE_DIM), lambda i: (i, 0))] + wspecs,
        out_specs=pl.BlockSpec((_EBLK, NODE_DIM), lambda i: (i, 0)),
        out_shape=jax.ShapeDtypeStruct((ep, NODE_DIM), jnp.float32),
        interpret=interpret,
    )(ev, xs, *wstacks)


def _build_wstacks(W_weight, W_bias):
    ws = {l3: [] for l3 in range(LMAX + 1)}
    bs = {l3: [] for l3 in range(LMAX + 1)}
    for p in _PATHS:
        l3 = p[2]
        off = _PATH_OFF[p]
        a = _ALPHA[l3]
        ws[l3].append(a * W_weight[0, off:off + MUL * MUL].reshape(MUL, MUL))
        bs[l3].append(a * W_bias[off:off + MUL * MUL].reshape(MUL, MUL))
    return [jnp.concatenate(ws[l3] + bs[l3], axis=0) for l3 in range(LMAX + 1)]


def kernel(node_features, edge_index, edge_vectors, W_weight, W_bias):
    N = node_features.shape[0]
    E = edge_index.shape[1]
    src = edge_index[0].astype(jnp.int32)
    dst = edge_index[1].astype(jnp.int32)

    xt = jnp.concatenate(
        [node_features[:, int(_OFFS[l]):int(_OFFS[l + 1])]
         .reshape(N, MUL, 2 * l + 1).swapaxes(1, 2).reshape(N, MUL * (2 * l + 1))
         for l in range(LMAX + 1)], axis=1)

    ep = ((E + _EBLK - 1) // _EBLK) * _EBLK
    src_p = jnp.concatenate([src, jnp.zeros((ep - E,), jnp.int32)])
    ev_p = jnp.concatenate([edge_vectors, jnp.zeros((ep - E, 3), jnp.float32)])

    xs = xt[src_p]
    wstacks = _build_wstacks(W_weight, W_bias)
    msg = _dense_call(ev_p, xs, wstacks)

    out_t = jnp.zeros((N, NODE_DIM), jnp.float32).at[dst].add(msg[:E])
    return out_t[:, jnp.asarray(_PERM_T2R)]


# speedup vs baseline: 2.3078x; 1.0369x over previous
import functools
from math import factorial, sqrt

import jax
import jax.numpy as jnp
import numpy as np
from jax import lax
from jax.experimental import pallas as pl
from jax.experimental.pallas import tpu as pltpu
from jax.experimental.pallas import tpu_sc as plsc

MUL = 32
LMAX = 3
IR_DIMS = [2 * l + 1 for l in range(LMAX + 1)]
NODE_DIM = MUL * sum(IR_DIMS)
_PATHS = [(0, 0, 0), (0, 1, 1), (0, 2, 2), (0, 3, 3), (1, 0, 1), (1, 1, 0),
          (1, 1, 2), (1, 2, 1), (1, 2, 3), (1, 3, 2), (2, 0, 2), (2, 1, 1),
          (2, 1, 3), (2, 2, 0), (2, 2, 2), (2, 3, 1), (2, 3, 3), (3, 0, 3),
          (3, 1, 2), (3, 2, 1), (3, 2, 3), (3, 3, 0), (3, 3, 2)]
_OFFS = np.cumsum([0] + [MUL * d for d in IR_DIMS])


def _cg(l1, m1, l2, m2, l3, m3):
    if m1 + m2 != m3:
        return 0.0
    pref = sqrt((2 * l3 + 1) * factorial(l3 + l1 - l2) * factorial(l3 - l1 + l2)
                * factorial(l1 + l2 - l3) / factorial(l1 + l2 + l3 + 1))
    pref *= sqrt(factorial(l3 + m3) * factorial(l3 - m3) * factorial(l1 - m1)
                 * factorial(l1 + m1) * factorial(l2 - m2) * factorial(l2 + m2))
    s = 0.0
    for k in range(0, l1 + l2 + l3 + 1):
        d = [k, l1 + l2 - l3 - k, l1 - m1 - k, l2 + m2 - k,
             l3 - l2 + m1 + k, l3 - l1 - m2 + k]
        if min(d) < 0:
            continue
        den = 1.0
        for t in d:
            den *= factorial(t)
        s += (-1) ** k / den
    return pref * s


def _w3j_real(l1, l2, l3):
    C = np.zeros((2 * l1 + 1, 2 * l2 + 1, 2 * l3 + 1), dtype=complex)
    for m1 in range(-l1, l1 + 1):
        for m2 in range(-l2, l2 + 1):
            for m3 in range(-l3, l3 + 1):
                C[l1 + m1, l2 + m2, l3 + m3] = _cg(l1, m1, l2, m2, l3, m3)

    def basis(l):
        U = np.zeros((2 * l + 1, 2 * l + 1), dtype=complex)
        s2 = 1.0 / sqrt(2.0)
        U[l, l] = 1.0
        for m in range(1, l + 1):
            U[l + m, l + m] = (-1) ** m * s2
            U[l + m, l - m] = s2
            U[l - m, l - m] = 1j * s2
            U[l - m, l + m] = -1j * ((-1) ** m) * s2
        return U

    U1, U2, U3 = basis(l1), basis(l2), basis(l3)
    Cr = np.einsum('am,bn,co,mno->abc', U1, U2, np.conj(U3), C)
    R = np.real(Cr)
    if np.linalg.norm(R) < 1e-6:
        R = np.imag(Cr)
    return (R / np.linalg.norm(R)).astype(np.float32)


_W3J = {p: _w3j_real(*p) for p in _PATHS}
_FANIN = {l3: MUL * sum(1 for p in _PATHS if p[2] == l3) for l3 in range(LMAX + 1)}
_ALPHA = {l3: float(np.sqrt((2 * l3 + 1) / _FANIN[l3])) for l3 in range(LMAX + 1)}
_PATHS_BY_L3 = {l3: [p for p in _PATHS if p[2] == l3] for l3 in range(LMAX + 1)}
_PATH_OFF = {p: 1024 * i for i, p in enumerate(_PATHS)}

_PERM_T2R = np.zeros(NODE_DIM, dtype=np.int32)
for _l in range(LMAX + 1):
    _d = 2 * _l + 1
    for _w in range(MUL):
        for _k in range(_d):
            _PERM_T2R[_OFFS[_l] + _w * _d + _k] = _OFFS[_l] + _k * MUL + _w

_EBLK = 256


def _dense_body(ev_ref, x_ref, w0_ref, w1_ref, w2_ref, w3_ref, out_ref):
    wrefs = [w0_ref, w1_ref, w2_ref, w3_ref]
    vx = ev_ref[:, 0:1]
    vy = ev_ref[:, 1:2]
    vz = ev_ref[:, 2:3]
    r2 = vx * vx + vy * vy + vz * vz
    length = jnp.maximum(jnp.sqrt(r2), 1e-8)
    inv = 1.0 / length
    x = vx * inv
    y = vy * inv
    z = vz * inv

    Y = {0: [jnp.ones_like(x)]}
    Y[1] = [sqrt(3.0) * y, sqrt(3.0) * z, sqrt(3.0) * x]
    Y[2] = [sqrt(15.0) * x * y, sqrt(15.0) * y * z,
            sqrt(5.0) / 2.0 * (3.0 * z * z - 1.0), sqrt(15.0) * x * z,
            sqrt(15.0) / 2.0 * (x * x - y * y)]
    Y[3] = [sqrt(35.0 / 8.0) * y * (3.0 * x * x - y * y),
            sqrt(105.0) * x * y * z,
            sqrt(21.0 / 8.0) * y * (5.0 * z * z - 1.0),
            sqrt(7.0) / 2.0 * (5.0 * z * z * z - 3.0 * z),
            sqrt(21.0 / 8.0) * x * (5.0 * z * z - 1.0),
            sqrt(105.0) / 2.0 * z * (x * x - y * y),
            sqrt(35.0 / 8.0) * x * (x * x - 3.0 * y * y)]

    def xsl(l, i):
        c = int(_OFFS[l]) + i * MUL
        return x_ref[:, c:c + MUL]

    for l3 in range(LMAX + 1):
        d3 = 2 * l3 + 1
        paths = _PATHS_BY_L3[l3]
        t_ks = []
        for k in range(d3):
            cols = []
            for (l1, l2, _l3) in paths:
                d1 = 2 * l1 + 1
                C = _W3J[(l1, l2, _l3)]
                acc = None
                for i in range(d1):
                    ycg = None
                    for j in range(2 * l2 + 1):
                        cv = float(C[i, j, k])
                        if cv == 0.0:
                            continue
                        term = cv * Y[l2][j]
                        ycg = term if ycg is None else ycg + term
                    if ycg is None:
                        continue
                    term = xsl(l1, i) * ycg
                    acc = term if acc is None else acc + term
                if acc is None:
                    acc = jnp.zeros((_EBLK, MUL), jnp.float32)
                cols.append(acc)
            t_ks.append(jnp.concatenate(cols, axis=1))
        tall = jnp.concatenate(t_ks, axis=0)
        lenf = jnp.concatenate([length] * d3, axis=0)
        tcat = jnp.concatenate([tall * lenf, tall], axis=1)
        o = lax.dot_general(tcat, wrefs[l3][...],
                            (((1,), (0,)), ((), ())),
                            preferred_element_type=jnp.float32)
        base = int(_OFFS[l3])
        for k in range(d3):
            out_ref[:, base + k * MUL:base + (k + 1) * MUL] = \
                o[k * _EBLK:(k + 1) * _EBLK, :]


def _dense_call(ev, xs, wstacks, interpret=False):
    ep = ev.shape[0]
    grid = ep // _EBLK
    wspecs = [pl.BlockSpec(w.shape, lambda i: (0, 0)) for w in wstacks]
    return pl.pallas_call(
        _dense_body,
        grid=(grid,),
        in_specs=[pl.BlockSpec((_EBLK, 3), lambda i: (i, 0)),
                  pl.BlockSpec((_EBLK, NODE_DIM), lambda i: (i, 0))] + wspecs,
        out_specs=pl.BlockSpec((_EBLK, NODE_DIM), lambda i: (i, 0)),
        out_shape=jax.ShapeDtypeStruct((ep, NODE_DIM), jnp.float32),
        interpret=interpret,
    )(ev, xs, *wstacks)


def _build_wstacks(W_weight, W_bias):
    ws = {l3: [] for l3 in range(LMAX + 1)}
    bs = {l3: [] for l3 in range(LMAX + 1)}
    for p in _PATHS:
        l3 = p[2]
        off = _PATH_OFF[p]
        a = _ALPHA[l3]
        ws[l3].append(a * W_weight[0, off:off + MUL * MUL].reshape(MUL, MUL))
        bs[l3].append(a * W_bias[off:off + MUL * MUL].reshape(MUL, MUL))
    return [jnp.concatenate(ws[l3] + bs[l3], axis=0) for l3 in range(LMAX + 1)]


_NC, _NS = 2, 16
_NW = _NC * _NS
_EP = 10240
_GCHUNK = 80
_SCHUNK = 64
_SNCH = _EP // _NS // _SCHUNK
_ACC_ROWS = 2048


def _gather_body(table_hbm, idx_hbm, out_hbm, idx_v, rows_v, sem):
    wid = lax.axis_index("s") * _NC + lax.axis_index("c")
    for c in range(4):
        base = wid * (4 * _GCHUNK) + c * _GCHUNK
        pltpu.sync_copy(idx_hbm.at[pl.ds(base, _GCHUNK)], idx_v)
        pltpu.async_copy(table_hbm.at[idx_v], rows_v, sem).wait()
        pltpu.sync_copy(rows_v, out_hbm.at[pl.ds(base, _GCHUNK)])


@functools.cache
def _gather_call():
    return pl.kernel(
        _gather_body,
        out_type=jax.ShapeDtypeStruct((_EP, NODE_DIM), jnp.float32),
        mesh=plsc.VectorSubcoreMesh(core_axis_name="c", subcore_axis_name="s"),
        scratch_types=[pltpu.VMEM((_GCHUNK,), jnp.int32),
                       pltpu.VMEM((_GCHUNK, NODE_DIM), jnp.float32),
                       pltpu.SemaphoreType.DMA],
    )


def _scatter_body(msg_hbm, dsti_hbm, z_hbm, out_hbm, idx_v, buf_v, acc_sh):
    cid = lax.axis_index("c")
    sid = lax.axis_index("s")

    @pl.when(cid == 0)
    def _():
        pltpu.sync_copy(z_hbm, acc_sh.at[pl.ds(sid * 128, 128)])
        plsc.subcore_barrier()
        for j in range(_SNCH):
            base = sid * (_SNCH * _SCHUNK) + j * _SCHUNK
            pltpu.sync_copy(dsti_hbm.at[pl.ds(base, _SCHUNK)], idx_v)
            pltpu.sync_copy(msg_hbm.at[pl.ds(base, _SCHUNK)], buf_v)
            pltpu.sync_copy(buf_v, acc_sh.at[idx_v], add=True)
        plsc.subcore_barrier()
        pltpu.sync_copy(acc_sh.at[pl.ds(sid * 128, 128)],
                        out_hbm.at[pl.ds(sid * 128, 128)])


@functools.cache
def _scatter_call():
    return pl.kernel(
        _scatter_body,
        out_type=jax.ShapeDtypeStruct((_ACC_ROWS, NODE_DIM), jnp.float32),
        mesh=plsc.VectorSubcoreMesh(core_axis_name="c", subcore_axis_name="s"),
        scratch_types=[pltpu.VMEM((_SCHUNK,), jnp.int32),
                       pltpu.VMEM((_SCHUNK, NODE_DIM), jnp.float32),
                       pltpu.VMEM_SHARED((_ACC_ROWS, NODE_DIM), jnp.float32)],
    )


def kernel(node_features, edge_index, edge_vectors, W_weight, W_bias):
    N = node_features.shape[0]
    E = edge_index.shape[1]
    src = edge_index[0].astype(jnp.int32)
    dst = edge_index[1].astype(jnp.int32)

    xt = jnp.concatenate(
        [node_features[:, int(_OFFS[l]):int(_OFFS[l + 1])]
         .reshape(N, MUL, 2 * l + 1).swapaxes(1, 2).reshape(N, MUL * (2 * l + 1))
         for l in range(LMAX + 1)], axis=1)

    ep = _EP
    src_p = jnp.concatenate([src, jnp.zeros((ep - E,), jnp.int32)])
    dst_p = jnp.concatenate(
        [dst, jnp.full((ep - E,), _ACC_ROWS - 1, jnp.int32)])
    ev_p = jnp.concatenate([edge_vectors, jnp.zeros((ep - E, 3), jnp.float32)])

    xs = _gather_call()(xt, src_p)
    wstacks = _build_wstacks(W_weight, W_bias)
    msg = _dense_call(ev_p, xs, wstacks)

    out_t = jnp.zeros((N, NODE_DIM), jnp.float32).at[dst].add(msg[:E])
    return out_t[:, jnp.asarray(_PERM_T2R)]


# speedup vs baseline: 7.3665x; 3.1920x over previous
import functools
from math import factorial, sqrt

import jax
import jax.numpy as jnp
import numpy as np
from jax import lax
from jax.experimental import pallas as pl
from jax.experimental.pallas import tpu as pltpu
from jax.experimental.pallas import tpu_sc as plsc

MUL = 32
LMAX = 3
IR_DIMS = [2 * l + 1 for l in range(LMAX + 1)]
NODE_DIM = MUL * sum(IR_DIMS)
_PATHS = [(0, 0, 0), (0, 1, 1), (0, 2, 2), (0, 3, 3), (1, 0, 1), (1, 1, 0),
          (1, 1, 2), (1, 2, 1), (1, 2, 3), (1, 3, 2), (2, 0, 2), (2, 1, 1),
          (2, 1, 3), (2, 2, 0), (2, 2, 2), (2, 3, 1), (2, 3, 3), (3, 0, 3),
          (3, 1, 2), (3, 2, 1), (3, 2, 3), (3, 3, 0), (3, 3, 2)]
_OFFS = np.cumsum([0] + [MUL * d for d in IR_DIMS])


def _cg(l1, m1, l2, m2, l3, m3):
    if m1 + m2 != m3:
        return 0.0
    pref = sqrt((2 * l3 + 1) * factorial(l3 + l1 - l2) * factorial(l3 - l1 + l2)
                * factorial(l1 + l2 - l3) / factorial(l1 + l2 + l3 + 1))
    pref *= sqrt(factorial(l3 + m3) * factorial(l3 - m3) * factorial(l1 - m1)
                 * factorial(l1 + m1) * factorial(l2 - m2) * factorial(l2 + m2))
    s = 0.0
    for k in range(0, l1 + l2 + l3 + 1):
        d = [k, l1 + l2 - l3 - k, l1 - m1 - k, l2 + m2 - k,
             l3 - l2 + m1 + k, l3 - l1 - m2 + k]
        if min(d) < 0:
            continue
        den = 1.0
        for t in d:
            den *= factorial(t)
        s += (-1) ** k / den
    return pref * s


def _w3j_real(l1, l2, l3):
    C = np.zeros((2 * l1 + 1, 2 * l2 + 1, 2 * l3 + 1), dtype=complex)
    for m1 in range(-l1, l1 + 1):
        for m2 in range(-l2, l2 + 1):
            for m3 in range(-l3, l3 + 1):
                C[l1 + m1, l2 + m2, l3 + m3] = _cg(l1, m1, l2, m2, l3, m3)

    def basis(l):
        U = np.zeros((2 * l + 1, 2 * l + 1), dtype=complex)
        s2 = 1.0 / sqrt(2.0)
        U[l, l] = 1.0
        for m in range(1, l + 1):
            U[l + m, l + m] = (-1) ** m * s2
            U[l + m, l - m] = s2
            U[l - m, l - m] = 1j * s2
            U[l - m, l + m] = -1j * ((-1) ** m) * s2
        return U

    U1, U2, U3 = basis(l1), basis(l2), basis(l3)
    Cr = np.einsum('am,bn,co,mno->abc', U1, U2, np.conj(U3), C)
    R = np.real(Cr)
    if np.linalg.norm(R) < 1e-6:
        R = np.imag(Cr)
    return (R / np.linalg.norm(R)).astype(np.float32)


_W3J = {p: _w3j_real(*p) for p in _PATHS}
_FANIN = {l3: MUL * sum(1 for p in _PATHS if p[2] == l3) for l3 in range(LMAX + 1)}
_ALPHA = {l3: float(np.sqrt((2 * l3 + 1) / _FANIN[l3])) for l3 in range(LMAX + 1)}
_PATHS_BY_L3 = {l3: [p for p in _PATHS if p[2] == l3] for l3 in range(LMAX + 1)}
_PATH_OFF = {p: 1024 * i for i, p in enumerate(_PATHS)}

_PERM_T2R = np.zeros(NODE_DIM, dtype=np.int32)
for _l in range(LMAX + 1):
    _d = 2 * _l + 1
    for _w in range(MUL):
        for _k in range(_d):
            _PERM_T2R[_OFFS[_l] + _w * _d + _k] = _OFFS[_l] + _k * MUL + _w

_EBLK = 256


def _dense_body(ev_ref, x_ref, w0_ref, w1_ref, w2_ref, w3_ref, out_ref):
    wrefs = [w0_ref, w1_ref, w2_ref, w3_ref]
    xt = x_ref[...].T
    vx = ev_ref[0:1, :]
    vy = ev_ref[1:2, :]
    vz = ev_ref[2:3, :]
    r2 = vx * vx + vy * vy + vz * vz
    length = jnp.maximum(jnp.sqrt(r2), 1e-8)
    inv = 1.0 / length
    x = vx * inv
    y = vy * inv
    z = vz * inv

    Y = {0: [jnp.ones_like(x)]}
    Y[1] = [sqrt(3.0) * y, sqrt(3.0) * z, sqrt(3.0) * x]
    Y[2] = [sqrt(15.0) * x * y, sqrt(15.0) * y * z,
            sqrt(5.0) / 2.0 * (3.0 * z * z - 1.0), sqrt(15.0) * x * z,
            sqrt(15.0) / 2.0 * (x * x - y * y)]
    Y[3] = [sqrt(35.0 / 8.0) * y * (3.0 * x * x - y * y),
            sqrt(105.0) * x * y * z,
            sqrt(21.0 / 8.0) * y * (5.0 * z * z - 1.0),
            sqrt(7.0) / 2.0 * (5.0 * z * z * z - 3.0 * z),
            sqrt(21.0 / 8.0) * x * (5.0 * z * z - 1.0),
            sqrt(105.0) / 2.0 * z * (x * x - y * y),
            sqrt(35.0 / 8.0) * x * (x * x - 3.0 * y * y)]

    def xsl(l, i):
        c = int(_OFFS[l]) + i * MUL
        return xt[c:c + MUL, :]

    for l3 in range(LMAX + 1):
        d3 = 2 * l3 + 1
        paths = _PATHS_BY_L3[l3]
        base = int(_OFFS[l3])
        for k in range(d3):
            rows = []
            for (l1, l2, _l3) in paths:
                d1 = 2 * l1 + 1
                C = _W3J[(l1, l2, _l3)]
                acc = None
                for i in range(d1):
                    ycg = None
                    for j in range(2 * l2 + 1):
                        cv = float(C[i, j, k])
                        if cv == 0.0:
                            continue
                        term = cv * Y[l2][j]
                        ycg = term if ycg is None else ycg + term
                    if ycg is None:
                        continue
                    term = xsl(l1, i) * ycg
                    acc = term if acc is None else acc + term
                if acc is None:
                    acc = jnp.zeros((MUL, _EBLK), jnp.float32)
                rows.append(acc)
            tcat = jnp.concatenate(rows, axis=0)
            tcat = jnp.concatenate([tcat * length, tcat], axis=0)
            o = lax.dot_general(wrefs[l3][...], tcat,
                                (((1,), (0,)), ((), ())),
                                preferred_element_type=jnp.float32)
            out_ref[base + k * MUL:base + (k + 1) * MUL, :] = o


def _dense_call(evt, xs, wstacks, interpret=False):
    ep = evt.shape[1]
    grid = ep // _EBLK
    wspecs = [pl.BlockSpec(w.shape, lambda i: (0, 0)) for w in wstacks]
    return pl.pallas_call(
        _dense_body,
        grid=(grid,),
        in_specs=[pl.BlockSpec((3, _EBLK), lambda i: (0, i)),
                  pl.BlockSpec((_EBLK, NODE_DIM), lambda i: (i, 0))] + wspecs,
        out_specs=pl.BlockSpec((NODE_DIM, _EBLK), lambda i: (0, i)),
        out_shape=jax.ShapeDtypeStruct((NODE_DIM, ep), jnp.float32),
        interpret=interpret,
    )(evt, xs, *wstacks)


def _build_wstacks(W_weight, W_bias):
    ws = {l3: [] for l3 in range(LMAX + 1)}
    bs = {l3: [] for l3 in range(LMAX + 1)}
    for p in _PATHS:
        l3 = p[2]
        off = _PATH_OFF[p]
        a = _ALPHA[l3]
        ws[l3].append(a * W_weight[0, off:off + MUL * MUL].reshape(MUL, MUL))
        bs[l3].append(a * W_bias[off:off + MUL * MUL].reshape(MUL, MUL))
    return [jnp.concatenate(ws[l3] + bs[l3], axis=0).T for l3 in range(LMAX + 1)]


_NC, _NS = 2, 16
_NW = _NC * _NS
_EP = 10240
_GCHUNK = 80
_SCHUNK = 64
_SNCH = _EP // _NS // _SCHUNK
_ACC_ROWS = 2048


def _gather_body(table_hbm, idx_hbm, out_hbm, idx_v, rows_v, sem):
    wid = lax.axis_index("s") * _NC + lax.axis_index("c")
    for c in range(4):
        base = wid * (4 * _GCHUNK) + c * _GCHUNK
        pltpu.sync_copy(idx_hbm.at[pl.ds(base, _GCHUNK)], idx_v)
        pltpu.async_copy(table_hbm.at[idx_v], rows_v, sem).wait()
        pltpu.sync_copy(rows_v, out_hbm.at[pl.ds(base, _GCHUNK)])


@functools.cache
def _gather_call():
    return pl.kernel(
        _gather_body,
        out_type=jax.ShapeDtypeStruct((_EP, NODE_DIM), jnp.float32),
        mesh=plsc.VectorSubcoreMesh(core_axis_name="c", subcore_axis_name="s"),
        scratch_types=[pltpu.VMEM((_GCHUNK,), jnp.int32),
                       pltpu.VMEM((_GCHUNK, NODE_DIM), jnp.float32),
                       pltpu.SemaphoreType.DMA],
    )


def _scatter_body(msg_hbm, dsti_hbm, z_hbm, out_hbm, idx_v, buf_v, acc_sh):
    cid = lax.axis_index("c")
    sid = lax.axis_index("s")

    @pl.when(cid == 0)
    def _():
        pltpu.sync_copy(z_hbm, acc_sh.at[pl.ds(sid * 128, 128)])
        plsc.subcore_barrier()
        for j in range(_SNCH):
            base = sid * (_SNCH * _SCHUNK) + j * _SCHUNK
            pltpu.sync_copy(dsti_hbm.at[pl.ds(base, _SCHUNK)], idx_v)
            pltpu.sync_copy(msg_hbm.at[pl.ds(base, _SCHUNK)], buf_v)
            pltpu.sync_copy(buf_v, acc_sh.at[idx_v], add=True)
        plsc.subcore_barrier()
        pltpu.sync_copy(acc_sh.at[pl.ds(sid * 128, 128)],
                        out_hbm.at[pl.ds(sid * 128, 128)])


@functools.cache
def _scatter_call():
    return pl.kernel(
        _scatter_body,
        out_type=jax.ShapeDtypeStruct((_ACC_ROWS, NODE_DIM), jnp.float32),
        mesh=plsc.VectorSubcoreMesh(core_axis_name="c", subcore_axis_name="s"),
        scratch_types=[pltpu.VMEM((_SCHUNK,), jnp.int32),
                       pltpu.VMEM((_SCHUNK, NODE_DIM), jnp.float32),
                       pltpu.VMEM_SHARED((_ACC_ROWS, NODE_DIM), jnp.float32)],
    )


def kernel(node_features, edge_index, edge_vectors, W_weight, W_bias):
    N = node_features.shape[0]
    E = edge_index.shape[1]
    src = edge_index[0].astype(jnp.int32)
    dst = edge_index[1].astype(jnp.int32)

    xt = jnp.concatenate(
        [node_features[:, int(_OFFS[l]):int(_OFFS[l + 1])]
         .reshape(N, MUL, 2 * l + 1).swapaxes(1, 2).reshape(N, MUL * (2 * l + 1))
         for l in range(LMAX + 1)], axis=1)

    ep = _EP
    src_p = jnp.concatenate([src, jnp.zeros((ep - E,), jnp.int32)])
    dst_p = jnp.concatenate(
        [dst, jnp.full((ep - E,), _ACC_ROWS - 1, jnp.int32)])
    evt_p = jnp.concatenate(
        [edge_vectors.T, jnp.zeros((3, ep - E), jnp.float32)], axis=1)

    xs = _gather_call()(xt, src_p)
    wstacks = _build_wstacks(W_weight, W_bias)
    msg = _dense_call(evt_p, xs, wstacks)

    out_t = jnp.zeros((N, NODE_DIM), jnp.float32).at[dst].add(msg.T[:E])
    return out_t[:, jnp.asarray(_PERM_T2R)]
